# baseline (speedup 1.0000x reference)
import jax
import jax.numpy as jnp
from jax.experimental import pallas as pl
from jax.experimental.pallas import tpu as pltpu

D_EMB = 1000
ROWS = 4096
COLS = 20
BLK = 128


def _zero_block(x_ref, o_ref):
    o_ref[...] = jnp.zeros((BLK, COLS, D_EMB), jnp.float32)


def kernel(x):
    return pl.pallas_call(
        _zero_block,
        grid=(ROWS // BLK,),
        in_specs=[pl.BlockSpec((BLK, COLS), lambda i: (i, 0))],
        out_specs=pl.BlockSpec((BLK, COLS, D_EMB), lambda i: (i, 0, 0)),
        out_shape=jax.ShapeDtypeStruct((ROWS, COLS, D_EMB), jnp.float32),
        compiler_params=pltpu.CompilerParams(
            dimension_semantics=("parallel",)),
    )(x)


# speedup vs baseline: 4.4633x; 4.4633x over previous
import jax
import jax.numpy as jnp
from jax.experimental import pallas as pl
from jax.experimental.pallas import tpu as pltpu

D_EMB = 1000
ROWS = 4096
COLS = 20
CBLK = 256


def _onehot_block(xt_ref, o_ref):
    xb = xt_ref[...]
    iota = jax.lax.broadcasted_iota(jnp.int32, (COLS, D_EMB, CBLK), 1)
    o_ref[...] = jnp.where(xb[:, None, :] == iota, 5.0, 0.0).astype(jnp.float32)


def kernel(x):
    xt = x.T
    out_t = pl.pallas_call(
        _onehot_block,
        grid=(ROWS // CBLK,),
        in_specs=[pl.BlockSpec((COLS, CBLK), lambda i: (0, i))],
        out_specs=pl.BlockSpec((COLS, D_EMB, CBLK), lambda i: (0, 0, i)),
        out_shape=jax.ShapeDtypeStruct((COLS, D_EMB, ROWS), jnp.float32),
        compiler_params=pltpu.CompilerParams(
            dimension_semantics=("parallel",)),
    )(xt)
    return out_t.transpose(2, 0, 1)
